# Initial kernel scaffold; baseline (speedup 1.0000x reference)
import jax
import jax.numpy as jnp
from jax.experimental import pallas as pl


def kernel(embedding, edges1, edges2, idx_mapping, W, att_src, att_dst, bias):
    raise NotImplementedError("write your pallas kernel here")


# speedup vs baseline: 22.8563x; 22.8563x over previous
import functools
import jax
import jax.numpy as jnp
from jax import lax
from jax.experimental import pallas as pl
from jax.experimental.pallas import tpu as pltpu
from jax.experimental.pallas import tpu_sc as plsc

N = 10000
D = 128
NPAD = 10240
NC = 2
NS = 16
NW = NC * NS
K = 128
BM = 512
R = NPAD // BM
RPT = NPAD // NS

_f32 = jnp.float32
_i32 = jnp.int32


def _mesh():
    return plsc.VectorSubcoreMesh(
        core_axis_name="c", subcore_axis_name="s", num_cores=NC, num_subcores=NS
    )


def _dense_first(x, W, att_src, att_dst):
    def body(x_ref, w_ref, asv_ref, adv_ref, h_ref, aso_ref, ado_ref):
        h = jnp.dot(x_ref[...], w_ref[...], preferred_element_type=_f32)
        h_ref[...] = h
        aso_ref[0, 0, :] = jnp.sum(h * asv_ref[0, :][None, :], axis=1)
        ado_ref[0, 0, :] = jnp.sum(h * adv_ref[0, :][None, :], axis=1)

    return pl.pallas_call(
        body,
        grid=(R,),
        in_specs=[
            pl.BlockSpec((BM, D), lambda i: (i, 0)),
            pl.BlockSpec((D, D), lambda i: (0, 0)),
            pl.BlockSpec((1, D), lambda i: (0, 0)),
            pl.BlockSpec((1, D), lambda i: (0, 0)),
        ],
        out_specs=[
            pl.BlockSpec((BM, D), lambda i: (i, 0)),
            pl.BlockSpec((1, 1, BM), lambda i: (i, 0, 0)),
            pl.BlockSpec((1, 1, BM), lambda i: (i, 0, 0)),
        ],
        out_shape=[
            jax.ShapeDtypeStruct((NPAD, D), _f32),
            jax.ShapeDtypeStruct((R, 1, BM), _f32),
            jax.ShapeDtypeStruct((R, 1, BM), _f32),
        ],
    )(x, W, att_src, att_dst)


def _dense_norm(num, s, W, att_src, att_dst, bias):
    def body(num_ref, s_ref, w_ref, asv_ref, adv_ref, b_ref,
             h_ref, aso_ref, ado_ref):
        acc = num_ref[0] + num_ref[1]
        den = s_ref[0, 0, :] + 1e-16
        x = acc / den[:, None] + b_ref[0, :][None, :]
        h = jnp.dot(x, w_ref[...], preferred_element_type=_f32)
        h_ref[...] = h
        aso_ref[0, 0, :] = jnp.sum(h * asv_ref[0, :][None, :], axis=1)
        ado_ref[0, 0, :] = jnp.sum(h * adv_ref[0, :][None, :], axis=1)

    return pl.pallas_call(
        body,
        grid=(R,),
        in_specs=[
            pl.BlockSpec((NC, BM, D), lambda i: (0, i, 0)),
            pl.BlockSpec((1, 1, BM), lambda i: (i, 0, 0)),
            pl.BlockSpec((D, D), lambda i: (0, 0)),
            pl.BlockSpec((1, D), lambda i: (0, 0)),
            pl.BlockSpec((1, D), lambda i: (0, 0)),
            pl.BlockSpec((1, D), lambda i: (0, 0)),
        ],
        out_specs=[
            pl.BlockSpec((BM, D), lambda i: (i, 0)),
            pl.BlockSpec((1, 1, BM), lambda i: (i, 0, 0)),
            pl.BlockSpec((1, 1, BM), lambda i: (i, 0, 0)),
        ],
        out_shape=[
            jax.ShapeDtypeStruct((NPAD, D), _f32),
            jax.ShapeDtypeStruct((R, 1, BM), _f32),
            jax.ShapeDtypeStruct((R, 1, BM), _f32),
        ],
    )(num, s, W, att_src, att_dst, bias)


def _norm_only(num, s, bias):
    def body(num_ref, s_ref, b_ref, x_ref):
        acc = num_ref[0] + num_ref[1]
        den = s_ref[0, 0, :] + 1e-16
        x_ref[...] = acc / den[:, None] + b_ref[0, :][None, :]

    return pl.pallas_call(
        body,
        grid=(R,),
        in_specs=[
            pl.BlockSpec((NC, BM, D), lambda i: (0, i, 0)),
            pl.BlockSpec((1, 1, BM), lambda i: (i, 0, 0)),
            pl.BlockSpec((1, D), lambda i: (0, 0)),
        ],
        out_specs=pl.BlockSpec((BM, D), lambda i: (i, 0)),
        out_shape=jax.ShapeDtypeStruct((NPAD, D), _f32),
    )(num, s, bias)


def _edge_stage(src2d, dst2d, a_s, a_d, h, n_blk):
    @functools.partial(
        pl.kernel,
        out_type=(
            jax.ShapeDtypeStruct((NC, NPAD, D), _f32),
            jax.ShapeDtypeStruct((NC, NPAD), _f32),
        ),
        mesh=_mesh(),
        compiler_params=pltpu.CompilerParams(needs_layout_passes=False),
        scratch_types=[
            pltpu.VMEM((n_blk, K), _i32),
            pltpu.VMEM((n_blk, K), _i32),
            pltpu.VMEM((K,), _f32),
            pltpu.VMEM((K,), _f32),
            pltpu.VMEM((K,), _f32),
            pltpu.VMEM((K, D), _f32),
            pltpu.VMEM((16, D), _f32),
            pltpu.VMEM((RPT,), _f32),
            pltpu.VMEM_SHARED((NPAD, D), _f32),
            pltpu.VMEM_SHARED((NPAD,), _f32),
            pltpu.VMEM_SHARED((NPAD,), _f32),
            pltpu.VMEM_SHARED((NPAD,), _f32),
            pltpu.SemaphoreType.DMA,
        ],
    )
    def k(src_hbm, dst_hbm, as_hbm, ad_hbm, h_hbm, num_out, s_out,
          src_v, dst_v, asg_v, adg_v, w_v, rows_v, zrow, svec,
          num_sh, s_sh, as_sh, ad_sh, sem):
        cid = lax.axis_index("c")
        sid = lax.axis_index("s")
        wid = sid * NC + cid
        row0 = sid * RPT

        pltpu.sync_copy(src_hbm.at[wid], src_v)
        pltpu.sync_copy(dst_hbm.at[wid], dst_v)
        pltpu.sync_copy(as_hbm.at[pl.ds(row0, RPT)], svec)
        pltpu.sync_copy(svec, as_sh.at[pl.ds(row0, RPT)])
        pltpu.sync_copy(ad_hbm.at[pl.ds(row0, RPT)], svec)
        pltpu.sync_copy(svec, ad_sh.at[pl.ds(row0, RPT)])

        zeros16 = jnp.zeros((16,), _f32)
        for r0 in range(16):
            for c0 in range(D // 16):
                zrow[r0, pl.ds(c0 * 16, 16)] = zeros16
        for c0 in range(RPT // 16):
            svec[pl.ds(c0 * 16, 16)] = zeros16

        def zloop(i, _):
            pltpu.sync_copy(zrow, num_sh.at[pl.ds(row0 + i * 16, 16)])
            return 0

        lax.fori_loop(0, RPT // 16, zloop, 0)
        pltpu.sync_copy(svec, s_sh.at[pl.ds(row0, RPT)])
        plsc.subcore_barrier()

        def blk(b, _):
            pltpu.async_copy(as_sh.at[src_v.at[b]], asg_v, sem).wait()
            pltpu.async_copy(ad_sh.at[dst_v.at[b]], adg_v, sem).wait()
            for c2 in range(K // 16):
                sl = pl.ds(c2 * 16, 16)
                e = asg_v[sl] + adg_v[sl]
                e = jnp.where(e >= 0.0, e, 0.2 * e)
                w_v[sl] = jnp.exp(e)
            pltpu.sync_copy(w_v, s_sh.at[dst_v.at[b]], add=True)
            pltpu.async_copy(h_hbm.at[src_v.at[b]], rows_v, sem).wait()

            def srow(j, _):
                wb = plsc.load_gather(w_v, [jnp.full((16,), j, _i32)])
                for c2 in range(D // 16):
                    sl = pl.ds(c2 * 16, 16)
                    rows_v[j, sl] = rows_v[j, sl] * wb
                return 0

            lax.fori_loop(0, K, srow, 0)
            pltpu.sync_copy(rows_v, num_sh.at[dst_v.at[b]], add=True)
            return 0

        lax.fori_loop(0, n_blk, blk, 0)
        plsc.subcore_barrier()

        def cploop(i, _):
            r = row0 + i * K
            pltpu.sync_copy(num_sh.at[pl.ds(r, K)], rows_v)
            pltpu.sync_copy(rows_v, num_out.at[cid, pl.ds(r, K)])
            return 0

        lax.fori_loop(0, RPT // K, cploop, 0)
        pltpu.sync_copy(s_sh.at[pl.ds(row0, RPT)], svec)
        pltpu.sync_copy(svec, s_out.at[cid, pl.ds(row0, RPT)])

    return k(src2d, dst2d, a_s, a_d, h)


def _final_gather(table, idx3d, bpw):
    nsub = bpw // K
    tot = NW * bpw

    @functools.partial(
        pl.kernel,
        out_type=jax.ShapeDtypeStruct((tot, D), _f32),
        mesh=_mesh(),
        scratch_types=[
            pltpu.VMEM((nsub, K), _i32),
            pltpu.VMEM((bpw, D), _f32),
            pltpu.SemaphoreType.DMA,
        ],
    )
    def k(tab, idx, out, idx_v, rows_v, sem):
        cid = lax.axis_index("c")
        sid = lax.axis_index("s")
        wid = sid * NC + cid
        pltpu.sync_copy(idx.at[wid], idx_v)
        for j in range(nsub):
            pltpu.async_copy(tab.at[idx_v.at[j]],
                             rows_v.at[pl.ds(j * K, K)], sem).wait()
        pltpu.sync_copy(rows_v, out.at[pl.ds(wid * bpw, bpw)])

    return k(table, idx3d)


def _prep_edges(edge_index):
    e = edge_index.astype(_i32)
    loops = jnp.arange(N, dtype=_i32)
    src = jnp.concatenate([e[0], loops])
    dst = jnp.concatenate([e[1], loops])
    tot = src.shape[0]
    n_blk = -(-tot // (NW * K))
    epad = n_blk * NW * K
    src = jnp.pad(src, (0, epad - tot))
    dst = jnp.pad(dst, (0, epad - tot), constant_values=N)
    return src.reshape(NW, n_blk, K), dst.reshape(NW, n_blk, K), n_blk


def kernel(embedding, edges1, edges2, idx_mapping, W, att_src, att_dst, bias):
    emb = jnp.pad(embedding, ((0, NPAD - N), (0, 0)))
    as2d = att_src.reshape(1, D)
    ad2d = att_dst.reshape(1, D)
    b2d = bias.reshape(1, D)

    s1e, d1e, nb1 = _prep_edges(edges1)
    s2e, d2e, nb2 = _prep_edges(edges2)

    h1, a_s1, a_d1 = _dense_first(emb, W, as2d, ad2d)
    num1, den1 = _edge_stage(s1e, d1e, a_s1.reshape(NPAD), a_d1.reshape(NPAD),
                             h1, nb1)
    s1r = (den1[0] + den1[1]).reshape(R, 1, BM)

    h2, a_s2, a_d2 = _dense_norm(num1, s1r, W, as2d, ad2d, b2d)
    num2, den2 = _edge_stage(s2e, d2e, a_s2.reshape(NPAD), a_d2.reshape(NPAD),
                             h2, nb2)
    s2r = (den2[0] + den2[1]).reshape(R, 1, BM)

    xf = _norm_only(num2, s2r, b2d)

    voc = idx_mapping.shape[0]
    bpw = K * (-(-voc // (NW * K)))
    idxp = jnp.pad(idx_mapping.astype(_i32), (0, NW * bpw - voc))
    out = _final_gather(xf, idxp.reshape(NW, bpw // K, K), bpw)
    return out[:voc]
